# baseline (speedup 1.0000x reference)
import functools

import jax
import jax.numpy as jnp
from jax import lax
from jax.experimental import pallas as pl
from jax.experimental.pallas import tpu as pltpu
from jax.experimental.pallas import tpu_sc as plsc

Q = 1000
QP = 1024
N = 4096
NC = 2
NS = 16
NW = NC * NS
CH = 32
RB = 512


@functools.lru_cache(maxsize=None)
def _make_sc_gather(V, D, B, ch, dtype_name):
    dtype = jnp.dtype(dtype_name)
    b_per_w = B // NW
    n_chunks = b_per_w // ch

    mesh = plsc.VectorSubcoreMesh(
        core_axis_name="c", subcore_axis_name="s",
        num_cores=NC, num_subcores=NS)

    @functools.partial(
        pl.kernel,
        out_type=jax.ShapeDtypeStruct((B, D), dtype),
        mesh=mesh,
        scratch_types=[
            pltpu.VMEM((n_chunks, ch), jnp.int32),
            pltpu.VMEM((ch, D), dtype),
            pltpu.VMEM((ch, D), dtype),
            pltpu.SemaphoreType.DMA,
            pltpu.SemaphoreType.DMA,
        ],
    )
    def gather(table_hbm, idx_hbm, out_hbm, idx_v, buf0, buf1, sem0, sem1):
        wid = lax.axis_index("s") * NC + lax.axis_index("c")
        base = wid * b_per_w
        pltpu.sync_copy(idx_hbm.at[wid], idx_v)
        bufs = (buf0, buf1)
        sems = (sem0, sem1)
        handles = [None] * n_chunks
        handles[0] = pltpu.async_copy(
            table_hbm.at[idx_v.at[0]], bufs[0], sems[0])
        for c in range(n_chunks):
            handles[c].wait()
            if c + 1 < n_chunks:
                handles[c + 1] = pltpu.async_copy(
                    table_hbm.at[idx_v.at[c + 1]],
                    bufs[(c + 1) % 2], sems[(c + 1) % 2])
            pltpu.sync_copy(bufs[c % 2],
                            out_hbm.at[pl.ds(base + c * ch, ch)])

    return gather


def _gram_body(g_ref, out_ref):
    i = pl.program_id(0)
    rows = g_ref[pl.ds(i * RB, RB), :]
    out_ref[...] = lax.dot_general(
        rows, g_ref[...], (((1,), (1,)), ((), ())),
        preferred_element_type=jnp.float32)


_gram = pl.pallas_call(
    _gram_body,
    grid=(N // RB,),
    in_specs=[pl.BlockSpec((N, QP), lambda i: (0, 0))],
    out_specs=pl.BlockSpec((RB, N), lambda i: (i, 0)),
    out_shape=jax.ShapeDtypeStruct((N, N), jnp.float32),
)


def kernel(view_indices, L):
    vi = view_indices.astype(jnp.int32)
    idx = vi.reshape(NW, N // NW // CH, CH)
    Lb = jnp.pad(L, ((0, 0), (0, QP - Q))).astype(jnp.bfloat16)
    Lw = lax.bitcast_convert_type(Lb.reshape(Q, QP // 2, 2), jnp.int32)
    Gw = _make_sc_gather(Q, QP // 2, N, CH, "int32")(Lw, idx)
    G = lax.bitcast_convert_type(Gw, jnp.bfloat16).reshape(N, QP)
    return _gram(G)


# speedup vs baseline: 1.9830x; 1.9830x over previous
import functools

import jax
import jax.numpy as jnp
from jax import lax
from jax.experimental import pallas as pl
from jax.experimental.pallas import tpu as pltpu
from jax.experimental.pallas import tpu_sc as plsc

Q = 1000
QP = 1024
N = 4096
NC = 2
NS = 16
NW = NC * NS
CH = 32
RB = 512


@functools.lru_cache(maxsize=None)
def _make_sc_gather(V, D, B, ch, dtype_name):
    dtype = jnp.dtype(dtype_name)
    b_per_w = B // NW
    n_chunks = b_per_w // ch

    mesh = plsc.VectorSubcoreMesh(
        core_axis_name="c", subcore_axis_name="s",
        num_cores=NC, num_subcores=NS)

    @functools.partial(
        pl.kernel,
        out_type=jax.ShapeDtypeStruct((B, D), dtype),
        mesh=mesh,
        scratch_types=[
            pltpu.VMEM((n_chunks, ch), jnp.int32),
            pltpu.VMEM((ch, D), dtype),
            pltpu.VMEM((ch, D), dtype),
            pltpu.SemaphoreType.DMA,
            pltpu.SemaphoreType.DMA,
        ],
    )
    def gather(table_hbm, idx_hbm, out_hbm, idx_v, buf0, buf1, sem0, sem1):
        wid = lax.axis_index("s") * NC + lax.axis_index("c")
        base = wid * b_per_w
        pltpu.sync_copy(idx_hbm.at[wid], idx_v)
        bufs = (buf0, buf1)
        sems = (sem0, sem1)
        handles = [None] * n_chunks
        handles[0] = pltpu.async_copy(
            table_hbm.at[idx_v.at[0]], bufs[0], sems[0])
        for c in range(n_chunks):
            handles[c].wait()
            if c + 1 < n_chunks:
                handles[c + 1] = pltpu.async_copy(
                    table_hbm.at[idx_v.at[c + 1]],
                    bufs[(c + 1) % 2], sems[(c + 1) % 2])
            pltpu.sync_copy(bufs[c % 2],
                            out_hbm.at[pl.ds(base + c * ch, ch)])

    return gather


def _gram_body(g_ref, out_ref, gb_ref):
    i = pl.program_id(0)

    @pl.when(i == 0)
    def _():
        gb_ref[...] = g_ref[...].astype(jnp.bfloat16)

    rows = gb_ref[pl.ds(i * RB, RB), :]
    out_ref[...] = lax.dot_general(
        rows, gb_ref[...], (((1,), (1,)), ((), ())),
        preferred_element_type=jnp.float32)


_gram = pl.pallas_call(
    _gram_body,
    grid=(N // RB,),
    in_specs=[pl.BlockSpec((N, QP), lambda i: (0, 0))],
    out_specs=pl.BlockSpec((RB, N), lambda i: (i, 0)),
    out_shape=jax.ShapeDtypeStruct((N, N), jnp.float32),
    scratch_shapes=[pltpu.VMEM((N, QP), jnp.bfloat16)],
)


def kernel(view_indices, L):
    vi = view_indices.astype(jnp.int32)
    idx = vi.reshape(NW, N // NW // CH, CH)
    Lp = jnp.pad(L, ((0, 0), (0, QP - Q)))
    G = _make_sc_gather(Q, QP, N, CH, "float32")(Lp, idx)
    return _gram(G)


# speedup vs baseline: 2.0201x; 1.0187x over previous
import functools

import jax
import jax.numpy as jnp
from jax import lax
from jax.experimental import pallas as pl
from jax.experimental.pallas import tpu as pltpu
from jax.experimental.pallas import tpu_sc as plsc

Q = 1000
QP = 1024
N = 4096
NC = 2
NS = 16
NW = NC * NS
CH = 32
RB = 512


@functools.lru_cache(maxsize=None)
def _make_sc_gather(V, D, B, ch, dtype_name):
    dtype = jnp.dtype(dtype_name)
    b_per_w = B // NW
    n_chunks = b_per_w // ch

    mesh = plsc.VectorSubcoreMesh(
        core_axis_name="c", subcore_axis_name="s",
        num_cores=NC, num_subcores=NS)

    @functools.partial(
        pl.kernel,
        out_type=jax.ShapeDtypeStruct((B, D), dtype),
        mesh=mesh,
        scratch_types=[
            pltpu.VMEM((b_per_w,), jnp.int32),
            pltpu.VMEM((ch, D), dtype),
            pltpu.VMEM((ch, D), dtype),
            pltpu.SemaphoreType.DMA,
            pltpu.SemaphoreType.DMA,
        ],
    )
    def gather(table_hbm, idx_hbm, out_hbm, idx_v, buf0, buf1, sem0, sem1):
        wid = lax.axis_index("s") * NC + lax.axis_index("c")
        base = wid * b_per_w
        pltpu.sync_copy(idx_hbm.at[pl.ds(base, b_per_w)], idx_v)
        bufs = (buf0, buf1)
        sems = (sem0, sem1)
        handles = [None] * n_chunks
        handles[0] = pltpu.async_copy(
            table_hbm.at[idx_v.at[pl.ds(0, ch)]], bufs[0], sems[0])
        for c in range(n_chunks):
            handles[c].wait()
            if c + 1 < n_chunks:
                handles[c + 1] = pltpu.async_copy(
                    table_hbm.at[idx_v.at[pl.ds((c + 1) * ch, ch)]],
                    bufs[(c + 1) % 2], sems[(c + 1) % 2])
            pltpu.sync_copy(bufs[c % 2],
                            out_hbm.at[pl.ds(base + c * ch, ch)])

    return gather


def _gram_body(g_ref, out_ref, gb_ref):
    i = pl.program_id(0)

    @pl.when(i == 0)
    def _():
        gb_ref[...] = g_ref[...].astype(jnp.bfloat16)

    rows = gb_ref[pl.ds(i * RB, RB), :]
    out_ref[...] = lax.dot_general(
        rows, gb_ref[...], (((1,), (1,)), ((), ())),
        preferred_element_type=jnp.float32)


_gram = pl.pallas_call(
    _gram_body,
    grid=(N // RB,),
    in_specs=[pl.BlockSpec((N, QP), lambda i: (0, 0))],
    out_specs=pl.BlockSpec((RB, N), lambda i: (i, 0)),
    out_shape=jax.ShapeDtypeStruct((N, N), jnp.float32),
    scratch_shapes=[pltpu.VMEM((N, QP), jnp.bfloat16)],
)


def kernel(view_indices, L):
    vi = view_indices.astype(jnp.int32)
    Lp = jnp.pad(L, ((0, 0), (0, QP - Q)))
    G = _make_sc_gather(Q, QP, N, CH, "float32")(Lp, vi)
    return _gram(G)


# speedup vs baseline: 2.0356x; 1.0077x over previous
import functools

import jax
import jax.numpy as jnp
from jax import lax
from jax.experimental import pallas as pl
from jax.experimental.pallas import tpu as pltpu
from jax.experimental.pallas import tpu_sc as plsc

Q = 1000
QP = 1024
N = 4096
NC = 2
NS = 16
NW = NC * NS
CH = 32
RB = 512


@functools.lru_cache(maxsize=None)
def _make_sc_gather(V, D, B, ch, dtype_name):
    dtype = jnp.dtype(dtype_name)
    b_per_w = B // NW
    n_chunks = b_per_w // ch

    mesh = plsc.VectorSubcoreMesh(
        core_axis_name="c", subcore_axis_name="s",
        num_cores=NC, num_subcores=NS)

    @functools.partial(
        pl.kernel,
        out_type=jax.ShapeDtypeStruct((B, D), dtype),
        mesh=mesh,
        scratch_types=[
            pltpu.VMEM((b_per_w,), jnp.int32),
            pltpu.VMEM((ch, D), dtype),
            pltpu.VMEM((ch, D), dtype),
            pltpu.SemaphoreType.DMA,
            pltpu.SemaphoreType.DMA,
        ],
    )
    def gather(table_hbm, idx_hbm, out_hbm, idx_v, buf0, buf1, sem0, sem1):
        wid = lax.axis_index("s") * NC + lax.axis_index("c")
        base = wid * b_per_w
        pltpu.sync_copy(idx_hbm.at[pl.ds(base, b_per_w)], idx_v)
        bufs = (buf0, buf1)
        sems = (sem0, sem1)
        handles = [None] * n_chunks
        handles[0] = pltpu.async_copy(
            table_hbm.at[idx_v.at[pl.ds(0, ch)]], bufs[0], sems[0])
        for c in range(n_chunks):
            handles[c].wait()
            if c + 1 < n_chunks:
                handles[c + 1] = pltpu.async_copy(
                    table_hbm.at[idx_v.at[pl.ds((c + 1) * ch, ch)]],
                    bufs[(c + 1) % 2], sems[(c + 1) % 2])
            pltpu.sync_copy(bufs[c % 2],
                            out_hbm.at[pl.ds(base + c * ch, ch)])

    return gather


def _gram_body(g_ref, out_ref):
    i = pl.program_id(0)
    rows = g_ref[pl.ds(i * RB, RB), :]
    out_ref[...] = lax.dot_general(
        rows, g_ref[...], (((1,), (1,)), ((), ())),
        preferred_element_type=jnp.float32)


_gram = pl.pallas_call(
    _gram_body,
    grid=(N // RB,),
    in_specs=[pl.BlockSpec((N, QP), lambda i: (0, 0))],
    out_specs=pl.BlockSpec((RB, N), lambda i: (i, 0)),
    out_shape=jax.ShapeDtypeStruct((N, N), jnp.float32),
)


def kernel(view_indices, L):
    vi = view_indices.astype(jnp.int32)
    Lp = jnp.pad(L, ((0, 0), (0, QP - Q)))
    G = _make_sc_gather(Q, QP, N, CH, "float32")(Lp, vi)
    return _gram(G)
